# baseline (speedup 1.0000x reference)
import functools

import jax
import jax.numpy as jnp
from jax import lax
from jax.experimental import pallas as pl
from jax.experimental.pallas import tpu as pltpu
from jax.experimental.pallas import tpu_sc as plsc

_L, _DIM, _B, _N, _K, _W = 2, 768, 1, 8192, 4, 128
_NW = _N // _W
_SCALE = 1.0 / (_DIM ** 0.5)
_NEG = -1e30

_BT = 512
_NB = _N // _BT
_WPB = _BT // _W

_SC_NC, _SC_NS = 2, 16
_SC_NWORK = _SC_NC * _SC_NS
_SC_CHUNK = 64


def _embed_gather(table, ids):
    rows_per_w = _N // _SC_NWORK
    nchunks = rows_per_w // _SC_CHUNK
    mesh = plsc.VectorSubcoreMesh(core_axis_name="c", subcore_axis_name="s")

    @functools.partial(
        pl.kernel,
        mesh=mesh,
        out_type=jax.ShapeDtypeStruct((_N, _DIM), jnp.float32),
        scratch_types=[
            pltpu.VMEM((_SC_CHUNK,), jnp.int32),
            pltpu.VMEM((_SC_CHUNK, _DIM), jnp.float32),
            pltpu.SemaphoreType.DMA,
        ],
    )
    def gather_kernel(table_hbm, idx_hbm, out_hbm, idx_v, rows_v, sem):
        wid = lax.axis_index("s") * _SC_NC + lax.axis_index("c")
        base = wid * rows_per_w
        for c in range(nchunks):
            off = base + c * _SC_CHUNK
            pltpu.sync_copy(idx_hbm.at[pl.ds(off, _SC_CHUNK)], idx_v)
            pltpu.async_copy(table_hbm.at[idx_v], rows_v, sem).wait()
            pltpu.sync_copy(rows_v, out_hbm.at[pl.ds(off, _SC_CHUNK)])

    return gather_kernel(table, ids)


def _ln(x, g):
    mu = jnp.mean(x, axis=1, keepdims=True)
    var = jnp.mean(x * x, axis=1, keepdims=True) - mu * mu
    return (x - mu) * lax.rsqrt(var + 1e-6) * g


def _top4_write(s, fi, idx_ref, val_ref):
    vals = []
    for j in range(_K):
        m = jnp.max(s)
        ix = jnp.min(jnp.where(s == m, fi, _N))
        idx_ref[j] = ix
        vals.append(jnp.reshape(m, (1, 1)))
        s = jnp.where(fi == ix, _NEG, s)
    val_ref[...] = jnp.concatenate(vals, axis=0)


def _attn_window(x, g, wq, wk, wv, wo, rq, rkv):
    def body(x_ref, g_ref, wq_ref, wk_ref, wv_ref, wo_ref, rq_ref, rkv_ref,
             y_ref, iq_ref, gq_ref, ikv_ref, gkv_ref, sq_s, skv_s):
        i = pl.program_id(0)
        xb = x_ref[...]
        h = _ln(xb, g_ref[...])
        h16 = h.astype(jnp.bfloat16)
        q16 = jnp.dot(h16, wq_ref[...], preferred_element_type=jnp.float32).astype(jnp.bfloat16)
        k16 = jnp.dot(h16, wk_ref[...], preferred_element_type=jnp.float32).astype(jnp.bfloat16)
        v16 = jnp.dot(h16, wv_ref[...], preferred_element_type=jnp.float32).astype(jnp.bfloat16)
        q3 = q16.reshape(_WPB, _W, _DIM)
        k3 = k16.reshape(_WPB, _W, _DIM)
        v3 = v16.reshape(_WPB, _W, _DIM)
        s3 = lax.dot_general(q3, k3, (((2,), (2,)), ((0,), (0,))),
                             preferred_element_type=jnp.float32) * _SCALE
        e3 = jnp.exp(s3)
        a3 = (e3 / jnp.sum(e3, axis=2, keepdims=True)).astype(jnp.bfloat16)
        av = lax.dot_general(a3, v3, (((2,), (1,)), ((0,), (0,))),
                             preferred_element_type=jnp.float32)
        av16 = av.astype(jnp.bfloat16).reshape(_BT, _DIM)
        y_ref[...] = xb + jnp.dot(av16, wo_ref[...], preferred_element_type=jnp.float32)
        sq_s[pl.ds(i, 1), :] = lax.dot_general(
            rq_ref[...], h, (((1,), (1,)), ((), ())),
            preferred_element_type=jnp.float32)
        skv_s[pl.ds(i, 1), :] = lax.dot_general(
            rkv_ref[...], h, (((1,), (1,)), ((), ())),
            preferred_element_type=jnp.float32)

        @pl.when(i == _NB - 1)
        def _topk():
            fi = (lax.broadcasted_iota(jnp.int32, (_NB, _BT), 0) * _BT
                  + lax.broadcasted_iota(jnp.int32, (_NB, _BT), 1))
            _top4_write(sq_s[...], fi, iq_ref, gq_ref)
            _top4_write(skv_s[...], fi, ikv_ref, gkv_ref)

    full = lambda shape: pl.BlockSpec(shape, lambda i: tuple(0 for _ in shape))
    return pl.pallas_call(
        body,
        grid=(_NB,),
        in_specs=[
            pl.BlockSpec((_BT, _DIM), lambda i: (i, 0)),
            full((1, _DIM)),
            full((_DIM, _DIM)), full((_DIM, _DIM)), full((_DIM, _DIM)), full((_DIM, _DIM)),
            full((1, _DIM)), full((1, _DIM)),
        ],
        out_specs=[
            pl.BlockSpec((_BT, _DIM), lambda i: (i, 0)),
            pl.BlockSpec(memory_space=pltpu.SMEM),
            full((_K, 1)),
            pl.BlockSpec(memory_space=pltpu.SMEM),
            full((_K, 1)),
        ],
        out_shape=[
            jax.ShapeDtypeStruct((_N, _DIM), jnp.float32),
            jax.ShapeDtypeStruct((_K,), jnp.int32),
            jax.ShapeDtypeStruct((_K, 1), jnp.float32),
            jax.ShapeDtypeStruct((_K,), jnp.int32),
            jax.ShapeDtypeStruct((_K, 1), jnp.float32),
        ],
        scratch_shapes=[pltpu.VMEM((_NB, _BT), jnp.float32)] * 2,
    )(x, g, wq, wk, wv, wo, rq, rkv)


def _ff_window(y, g, w1, w2, rff):
    def body(y_ref, g_ref, w1_ref, w2_ref, rff_ref,
             z_ref, iff_ref, gff_ref, sff_s):
        i = pl.program_id(0)
        xb = y_ref[...]
        h = _ln(xb, g_ref[...])
        hh = jnp.maximum(jnp.dot(h.astype(jnp.bfloat16), w1_ref[...],
                                 preferred_element_type=jnp.float32), 0.0)
        z_ref[...] = xb + jnp.dot(hh.astype(jnp.bfloat16), w2_ref[...],
                                  preferred_element_type=jnp.float32)
        sff_s[pl.ds(i, 1), :] = lax.dot_general(
            rff_ref[...], h, (((1,), (1,)), ((), ())),
            preferred_element_type=jnp.float32)

        @pl.when(i == _NB - 1)
        def _topk():
            fi = (lax.broadcasted_iota(jnp.int32, (_NB, _BT), 0) * _BT
                  + lax.broadcasted_iota(jnp.int32, (_NB, _BT), 1))
            _top4_write(sff_s[...], fi, iff_ref, gff_ref)

    full = lambda shape: pl.BlockSpec(shape, lambda i: tuple(0 for _ in shape))
    return pl.pallas_call(
        body,
        grid=(_NB,),
        in_specs=[
            pl.BlockSpec((_BT, _DIM), lambda i: (i, 0)),
            full((1, _DIM)),
            full((_DIM, _DIM // 2)), full((_DIM // 2, _DIM)),
            full((1, _DIM)),
        ],
        out_specs=[
            pl.BlockSpec((_BT, _DIM), lambda i: (i, 0)),
            pl.BlockSpec(memory_space=pltpu.SMEM),
            full((_K, 1)),
        ],
        out_shape=[
            jax.ShapeDtypeStruct((_N, _DIM), jnp.float32),
            jax.ShapeDtypeStruct((_K,), jnp.int32),
            jax.ShapeDtypeStruct((_K, 1), jnp.float32),
        ],
        scratch_shapes=[pltpu.VMEM((_NB, _BT), jnp.float32)],
    )(y, g, w1, w2, rff)


def _heavy_attn(xin, gq, gkv, g, wq, wk, wv, wo, idx_all):
    xin = xin.reshape(_N, 1, _DIM)

    def body(pidx_ref, *refs):
        row_refs = refs[:2 * _K]
        gq_ref, gkv_ref, g_ref, wq_ref, wk_ref, wv_ref, wo_ref, out_ref = refs[2 * _K:]
        rows_q = jnp.concatenate([row_refs[r][...].reshape(1, _DIM) for r in range(_K)], axis=0)
        rows_k = jnp.concatenate([row_refs[_K + r][...].reshape(1, _DIM) for r in range(_K)], axis=0)
        hq = _ln(rows_q, g_ref[...]).astype(jnp.bfloat16)
        hkv = _ln(rows_k, g_ref[...]).astype(jnp.bfloat16)
        qh = jnp.dot(hq, wq_ref[...], preferred_element_type=jnp.float32)
        kh = jnp.dot(hkv, wk_ref[...], preferred_element_type=jnp.float32)
        vh = jnp.dot(hkv, wv_ref[...], preferred_element_type=jnp.float32)
        vh = vh * jax.nn.sigmoid(gkv_ref[...])
        s = lax.dot_general(qh.astype(jnp.bfloat16), kh.astype(jnp.bfloat16),
                            (((1,), (1,)), ((), ())),
                            preferred_element_type=jnp.float32) * _SCALE
        e = jnp.exp(s)
        a = e / jnp.sum(e, axis=1, keepdims=True)
        oh = jnp.dot(jnp.dot(a, vh, preferred_element_type=jnp.float32).astype(jnp.bfloat16),
                     wo_ref[...], preferred_element_type=jnp.float32)
        out_ref[...] = oh * jax.nn.sigmoid(gq_ref[...])

    def row_spec(j):
        return pl.BlockSpec((1, 1, _DIM), lambda i, p, _j=j: (p[_j], 0, 0))

    full = lambda shape: pl.BlockSpec(shape, lambda i, p: tuple(0 for _ in shape))
    gs = pltpu.PrefetchScalarGridSpec(
        num_scalar_prefetch=1,
        grid=(1,),
        in_specs=[row_spec(j) for j in range(2 * _K)] + [
            full((_K, 1)), full((_K, 1)), full((1, _DIM)),
            full((_DIM, _DIM)), full((_DIM, _DIM)), full((_DIM, _DIM)), full((_DIM, _DIM)),
        ],
        out_specs=full((_K, _DIM)),
    )
    return pl.pallas_call(
        body,
        grid_spec=gs,
        out_shape=jax.ShapeDtypeStruct((_K, _DIM), jnp.float32),
    )(idx_all, *([xin] * (2 * _K)), gq, gkv, g, wq, wk, wv, wo)


def _heavy_ff(yin, gff, g, w1, w2, idx_all):
    yin = yin.reshape(_N, 1, _DIM)

    def body(pidx_ref, *refs):
        row_refs = refs[:_K]
        gff_ref, g_ref, w1_ref, w2_ref, out_ref = refs[_K:]
        rows = jnp.concatenate([row_refs[r][...].reshape(1, _DIM) for r in range(_K)], axis=0)
        h = _ln(rows, g_ref[...]).astype(jnp.bfloat16)
        hh = jnp.maximum(jnp.dot(h, w1_ref[...], preferred_element_type=jnp.float32), 0.0)
        out = jnp.dot(hh.astype(jnp.bfloat16), w2_ref[...],
                      preferred_element_type=jnp.float32)
        out_ref[...] = out * jax.nn.sigmoid(gff_ref[...])

    def row_spec(j):
        return pl.BlockSpec((1, 1, _DIM), lambda i, p, _j=j: (p[_j], 0, 0))

    full = lambda shape: pl.BlockSpec(shape, lambda i, p: tuple(0 for _ in shape))
    gs = pltpu.PrefetchScalarGridSpec(
        num_scalar_prefetch=1,
        grid=(1,),
        in_specs=[row_spec(j) for j in range(_K)] + [
            full((_K, 1)), full((1, _DIM)),
            full((_DIM, 4 * _DIM)), full((4 * _DIM, _DIM)),
        ],
        out_specs=full((_K, _DIM)),
    )
    return pl.pallas_call(
        body,
        grid_spec=gs,
        out_shape=jax.ShapeDtypeStruct((_K, _DIM), jnp.float32),
    )(idx_all, *([yin] * _K), gff, g, w1, w2)


def _apply_scatter(base, oh, idx):
    def body(pidx_ref, b_ref, oh_ref, out_ref):
        i = pl.program_id(0)
        riota = lax.broadcasted_iota(jnp.int32, (_K, 1), 0)
        row = jnp.sum((riota == i).astype(jnp.float32) * oh_ref[...], axis=0,
                      keepdims=True)
        out_ref[...] = b_ref[...] + row.reshape(1, 1, _DIM)

    gs = pltpu.PrefetchScalarGridSpec(
        num_scalar_prefetch=1,
        grid=(_K,),
        in_specs=[
            pl.BlockSpec((1, 1, _DIM), lambda i, p: (p[i], 0, 0)),
            pl.BlockSpec((_K, _DIM), lambda i, p: (0, 0)),
        ],
        out_specs=pl.BlockSpec((1, 1, _DIM), lambda i, p: (p[i], 0, 0)),
    )
    out = pl.pallas_call(
        body,
        grid_spec=gs,
        out_shape=jax.ShapeDtypeStruct((_N, 1, _DIM), jnp.float32),
        input_output_aliases={1: 0},
    )(idx, base.reshape(_N, 1, _DIM), oh)
    return out.reshape(_N, _DIM)


def kernel(input_ids, embed, ln1_g, ln2_g, Wq_l, Wk_l, Wv_l, Wo_l,
           Wq_h, Wk_h, Wv_h, Wo_h, r_q, r_kv, r_ff,
           ff_l_w1, ff_l_w2, ff_h_w1, ff_h_w2):
    ids = input_ids.reshape(_N).astype(jnp.int32)
    x = _embed_gather(embed, ids)

    bf = jnp.bfloat16
    Wq_l16, Wk_l16, Wv_l16, Wo_l16 = (w.astype(bf) for w in (Wq_l, Wk_l, Wv_l, Wo_l))
    Wq_h16, Wk_h16, Wv_h16, Wo_h16 = (w.astype(bf) for w in (Wq_h, Wk_h, Wv_h, Wo_h))
    w1l16, w2l16 = ff_l_w1.astype(bf), ff_l_w2.astype(bf)
    w1h16, w2h16 = ff_h_w1.astype(bf), ff_h_w2.astype(bf)

    for l in range(_L):
        g1 = ln1_g[l].reshape(1, _DIM)
        g2 = ln2_g[l].reshape(1, _DIM)
        rq = r_q[l].reshape(1, _DIM)
        rkv = r_kv[l].reshape(1, _DIM)
        rff = r_ff[l].reshape(1, _DIM)

        y, iq, gq, ikv, gkv = _attn_window(x, g1, Wq_l16[l], Wk_l16[l],
                                           Wv_l16[l], Wo_l16[l], rq, rkv)
        oh_a = _heavy_attn(x, gq, gkv, g1, Wq_h16[l], Wk_h16[l], Wv_h16[l],
                           Wo_h16[l], jnp.concatenate([iq, ikv]))
        y = _apply_scatter(y, oh_a, iq)
        z, iff, gff = _ff_window(y, g2, w1l16[l], w2l16[l], rff)
        oh_f = _heavy_ff(y, gff, g2, w1h16[l], w2h16[l], iff)
        x = _apply_scatter(z, oh_f, iff)

    return x.reshape(_B, _N, _DIM)
